# baseline (speedup 1.0000x reference)
import jax
import jax.numpy as jnp
from jax.experimental import pallas as pl
from jax.experimental.pallas import tpu as pltpu

_B, _V, _F = 8, 2048, 64
_K = 16
_NDIM = 4
_NPROP = 64
_NFILT = 128
_EXPF = 10.0
_R = 256
_NEG = float("-inf")


def _xform_body(x_ref, Ws_ref, bs_ref, Wf_ref, bf_ref, c_ref, f_ref):
    xb = x_ref[0].astype(jnp.bfloat16)
    c_ref[0] = (jnp.dot(xb, Ws_ref[...].astype(jnp.bfloat16),
                        preferred_element_type=jnp.float32) + bs_ref[...])
    f_ref[0] = (jnp.dot(xb, Wf_ref[...].astype(jnp.bfloat16),
                        preferred_element_type=jnp.float32) + bf_ref[...])


def _main_body(x_rows_ref, c_rows_ref, c_all_ref, sqc_ref, sqr_ref,
               feats_ref, Wo_ref, bo_ref, out_ref):
    x_rows = x_rows_ref[0]
    c_rows = c_rows_ref[0]
    c_all = c_all_ref[0]
    sq_col = sqc_ref[0][:, 0:1]
    sq_row = sqr_ref[0][0:1, :]
    g = jax.lax.dot_general(c_rows.astype(jnp.bfloat16),
                            c_all.astype(jnp.bfloat16),
                            (((1,), (1,)), ((), ())),
                            preferred_element_type=jnp.float32)
    dist = jnp.abs((-2.0 * g + sq_col) + sq_row)
    negd = -dist
    feats_b = feats_ref[0].astype(jnp.bfloat16)
    sumacc = jnp.zeros((_R, _NPROP), jnp.float32)
    maxacc = jnp.full((_R, _NPROP), _NEG, jnp.float32)
    taken = jnp.zeros((_R, 1), jnp.float32)
    m = jnp.max(negd, axis=1, keepdims=True)
    for k in range(_K):
        onehot = jnp.where(negd == m, 1.0, 0.0)
        negd = negd - onehot * 1e30
        m_next = jnp.max(negd, axis=1, keepdims=True)
        gke = jnp.dot(onehot.astype(jnp.bfloat16), feats_b,
                      preferred_element_type=jnp.float32)
        if k == 0:
            taken = gke[:, _NPROP:_NPROP + 1] - 1.0
        else:
            gate = taken < float(_K - 1)
            taken = taken + gke[:, _NPROP:_NPROP + 1]
            w = jnp.exp(_EXPF * m)
            wk = w * gke[:, 0:_NPROP]
            sumacc = sumacc + jnp.where(gate, wk, 0.0)
            maxacc = jnp.maximum(maxacc, jnp.where(gate, wk, _NEG))
        m = m_next
    mean = sumacc * (1.0 / (_K - 1))
    Wo = Wo_ref[...]
    out = (jnp.dot(x_rows.astype(jnp.bfloat16),
                   Wo[0:_F].astype(jnp.bfloat16),
                   preferred_element_type=jnp.float32)
           + jnp.dot(mean.astype(jnp.bfloat16),
                     Wo[_F:_F + _NPROP].astype(jnp.bfloat16),
                     preferred_element_type=jnp.float32)
           + jnp.dot(maxacc.astype(jnp.bfloat16),
                     Wo[_F + _NPROP:].astype(jnp.bfloat16),
                     preferred_element_type=jnp.float32)
           + bo_ref[...])
    out_ref[0] = out


def kernel(x, W_flr, b_flr, W_s, b_s, W_out, b_out):
    Ws_pad = jnp.zeros((_F, 128), jnp.float32).at[:, :_NDIM].set(W_s)
    bs_pad = jnp.zeros((1, 128), jnp.float32).at[:, :_NDIM].set(b_s)
    Wf_ext = jnp.zeros((_F, 128), jnp.float32).at[:, :_NPROP].set(W_flr)
    bf_ext = (jnp.zeros((1, 128), jnp.float32)
              .at[:, :_NPROP].set(b_flr.reshape(1, _NPROP))
              .at[:, _NPROP].set(1.0))
    bo = b_out.reshape(1, _NFILT)

    cpad, feats = pl.pallas_call(
        _xform_body,
        grid=(_B,),
        in_specs=[
            pl.BlockSpec((1, _V, _F), lambda b: (b, 0, 0)),
            pl.BlockSpec((_F, 128), lambda b: (0, 0)),
            pl.BlockSpec((1, 128), lambda b: (0, 0)),
            pl.BlockSpec((_F, 128), lambda b: (0, 0)),
            pl.BlockSpec((1, 128), lambda b: (0, 0)),
        ],
        out_specs=[
            pl.BlockSpec((1, _V, 128), lambda b: (b, 0, 0)),
            pl.BlockSpec((1, _V, 128), lambda b: (b, 0, 0)),
        ],
        out_shape=[
            jax.ShapeDtypeStruct((_B, _V, 128), jnp.float32),
            jax.ShapeDtypeStruct((_B, _V, 128), jnp.float32),
        ],
        compiler_params=pltpu.CompilerParams(
            dimension_semantics=("parallel",)),
    )(x, Ws_pad, bs_pad, Wf_ext, bf_ext)

    csl = cpad[:, :, :_NDIM]
    sq = jnp.sum(csl * csl, axis=2)
    sq_col = jnp.broadcast_to(sq[:, :, None], (_B, _V, 8))
    sq_row = jnp.broadcast_to(sq[:, None, :], (_B, 8, _V))

    return pl.pallas_call(
        _main_body,
        grid=(_B, _V // _R),
        in_specs=[
            pl.BlockSpec((1, _R, _F), lambda b, r: (b, r, 0)),
            pl.BlockSpec((1, _R, 128), lambda b, r: (b, r, 0)),
            pl.BlockSpec((1, _V, 128), lambda b, r: (b, 0, 0)),
            pl.BlockSpec((1, _R, 8), lambda b, r: (b, r, 0)),
            pl.BlockSpec((1, 8, _V), lambda b, r: (b, 0, 0)),
            pl.BlockSpec((1, _V, 128), lambda b, r: (b, 0, 0)),
            pl.BlockSpec((_F + 2 * _NPROP, _NFILT), lambda b, r: (0, 0)),
            pl.BlockSpec((1, _NFILT), lambda b, r: (0, 0)),
        ],
        out_specs=pl.BlockSpec((1, _R, _NFILT), lambda b, r: (b, r, 0)),
        out_shape=jax.ShapeDtypeStruct((_B, _V, _NFILT), jnp.float32),
        compiler_params=pltpu.CompilerParams(
            dimension_semantics=("parallel", "arbitrary")),
    )(x, cpad, cpad, sq_col, sq_row, feats, W_out, bo)


# speedup vs baseline: 1.2023x; 1.2023x over previous
import jax
import jax.numpy as jnp
from jax.experimental import pallas as pl
from jax.experimental.pallas import tpu as pltpu

_B, _V, _F = 8, 2048, 64
_K = 16
_NDIM = 4
_NPROP = 64
_NFILT = 128
_EXPF = 10.0
_R = 256
_NEG = float("-inf")


def _xform_body(x_ref, Ws_ref, bs_ref, Wf_ref, bf_ref, c_ref, f_ref):
    xb = x_ref[0].astype(jnp.bfloat16)
    c_ref[0] = (jnp.dot(xb, Ws_ref[...].astype(jnp.bfloat16),
                        preferred_element_type=jnp.float32) + bs_ref[...])
    f_ref[0] = (jnp.dot(xb, Wf_ref[...].astype(jnp.bfloat16),
                        preferred_element_type=jnp.float32) + bf_ref[...])


def _main_body(x_rows_ref, c_rows_ref, c_all_ref, sqc_ref, sqr_ref,
               feats_ref, Wo_ref, bo_ref, out_ref):
    x_rows = x_rows_ref[0]
    c_rows = c_rows_ref[0]
    c_all = c_all_ref[0]
    sq_col = sqc_ref[0][:, 0:1]
    sq_row = sqr_ref[0][0:1, :]
    g = jax.lax.dot_general(c_rows.astype(jnp.bfloat16),
                            c_all.astype(jnp.bfloat16),
                            (((1,), (1,)), ((), ())),
                            preferred_element_type=jnp.float32)
    dist = jnp.abs((-2.0 * g + sq_col) + sq_row)
    negd = -dist
    feats = feats_ref[0]
    sumacc = jnp.zeros((_R, _NPROP), jnp.float32)
    maxacc = jnp.full((_R, _NPROP), _NEG, jnp.float32)
    taken = jnp.zeros((_R, 1), jnp.float32)
    m = jnp.max(negd, axis=1, keepdims=True)
    for k in range(_K):
        onehot = jnp.where(negd == m, 1.0, 0.0)
        negd = negd - onehot * 1e30
        m_next = jnp.max(negd, axis=1, keepdims=True)
        gke = jnp.dot(onehot, feats,
                      preferred_element_type=jnp.float32)
        if k == 0:
            taken = gke[:, _NPROP:_NPROP + 1] - 1.0
        else:
            gate = taken < float(_K - 1)
            taken = taken + gke[:, _NPROP:_NPROP + 1]
            w = jnp.exp(_EXPF * m)
            wk = w * gke[:, 0:_NPROP]
            sumacc = sumacc + jnp.where(gate, wk, 0.0)
            maxacc = jnp.maximum(maxacc, jnp.where(gate, wk, _NEG))
        m = m_next
    mean = sumacc * (1.0 / (_K - 1))
    Wo = Wo_ref[...]
    out = (jnp.dot(x_rows.astype(jnp.bfloat16),
                   Wo[0:_F].astype(jnp.bfloat16),
                   preferred_element_type=jnp.float32)
           + jnp.dot(mean.astype(jnp.bfloat16),
                     Wo[_F:_F + _NPROP].astype(jnp.bfloat16),
                     preferred_element_type=jnp.float32)
           + jnp.dot(maxacc.astype(jnp.bfloat16),
                     Wo[_F + _NPROP:].astype(jnp.bfloat16),
                     preferred_element_type=jnp.float32)
           + bo_ref[...])
    out_ref[0] = out


def kernel(x, W_flr, b_flr, W_s, b_s, W_out, b_out):
    Ws_pad = jnp.zeros((_F, 128), jnp.float32).at[:, :_NDIM].set(W_s)
    bs_pad = jnp.zeros((1, 128), jnp.float32).at[:, :_NDIM].set(b_s)
    Wf_ext = jnp.zeros((_F, 128), jnp.float32).at[:, :_NPROP].set(W_flr)
    bf_ext = (jnp.zeros((1, 128), jnp.float32)
              .at[:, :_NPROP].set(b_flr.reshape(1, _NPROP))
              .at[:, _NPROP].set(1.0))
    bo = b_out.reshape(1, _NFILT)

    cpad, feats = pl.pallas_call(
        _xform_body,
        grid=(_B,),
        in_specs=[
            pl.BlockSpec((1, _V, _F), lambda b: (b, 0, 0)),
            pl.BlockSpec((_F, 128), lambda b: (0, 0)),
            pl.BlockSpec((1, 128), lambda b: (0, 0)),
            pl.BlockSpec((_F, 128), lambda b: (0, 0)),
            pl.BlockSpec((1, 128), lambda b: (0, 0)),
        ],
        out_specs=[
            pl.BlockSpec((1, _V, 128), lambda b: (b, 0, 0)),
            pl.BlockSpec((1, _V, 128), lambda b: (b, 0, 0)),
        ],
        out_shape=[
            jax.ShapeDtypeStruct((_B, _V, 128), jnp.float32),
            jax.ShapeDtypeStruct((_B, _V, 128), jnp.float32),
        ],
        compiler_params=pltpu.CompilerParams(
            dimension_semantics=("parallel",)),
    )(x, Ws_pad, bs_pad, Wf_ext, bf_ext)

    csl = cpad[:, :, :_NDIM]
    sq = jnp.sum(csl * csl, axis=2)
    sq_col = jnp.broadcast_to(sq[:, :, None], (_B, _V, 8))
    sq_row = jnp.broadcast_to(sq[:, None, :], (_B, 8, _V))

    return pl.pallas_call(
        _main_body,
        grid=(_B, _V // _R),
        in_specs=[
            pl.BlockSpec((1, _R, _F), lambda b, r: (b, r, 0)),
            pl.BlockSpec((1, _R, 128), lambda b, r: (b, r, 0)),
            pl.BlockSpec((1, _V, 128), lambda b, r: (b, 0, 0)),
            pl.BlockSpec((1, _R, 8), lambda b, r: (b, r, 0)),
            pl.BlockSpec((1, 8, _V), lambda b, r: (b, 0, 0)),
            pl.BlockSpec((1, _V, 128), lambda b, r: (b, 0, 0)),
            pl.BlockSpec((_F + 2 * _NPROP, _NFILT), lambda b, r: (0, 0)),
            pl.BlockSpec((1, _NFILT), lambda b, r: (0, 0)),
        ],
        out_specs=pl.BlockSpec((1, _R, _NFILT), lambda b, r: (b, r, 0)),
        out_shape=jax.ShapeDtypeStruct((_B, _V, _NFILT), jnp.float32),
        compiler_params=pltpu.CompilerParams(
            dimension_semantics=("parallel", "arbitrary")),
    )(x, cpad, cpad, sq_col, sq_row, feats, W_out, bo)


# speedup vs baseline: 1.3843x; 1.1513x over previous
import jax
import jax.numpy as jnp
from jax.experimental import pallas as pl
from jax.experimental.pallas import tpu as pltpu

_B, _V, _F = 8, 2048, 64
_K = 16
_NDIM = 4
_NPROP = 64
_NFILT = 128
_EXPF = 10.0
_R = 512
_NEG = float("-inf")


def _xform_body(x_ref, Ws_ref, bs_ref, Wf_ref, bf_ref, c_ref, f_ref):
    xb = x_ref[0].astype(jnp.bfloat16)
    c_ref[0] = (jnp.dot(xb, Ws_ref[...].astype(jnp.bfloat16),
                        preferred_element_type=jnp.float32) + bs_ref[...])
    f_ref[0] = (jnp.dot(xb, Wf_ref[...].astype(jnp.bfloat16),
                        preferred_element_type=jnp.float32) + bf_ref[...])


def _main_body(x_rows_ref, c_rows_ref, c_all_ref, sqc_ref, sqr_ref,
               feats_ref, Wo_ref, bo_ref, out_ref):
    x_rows = x_rows_ref[0]
    c_rows = c_rows_ref[0]
    c_all = c_all_ref[0]
    sq_col = sqc_ref[0][:, 0:1]
    sq_row = sqr_ref[0][0:1, :]
    g = jax.lax.dot_general(c_rows.astype(jnp.bfloat16),
                            c_all.astype(jnp.bfloat16),
                            (((1,), (1,)), ((), ())),
                            preferred_element_type=jnp.float32)
    dist = jnp.abs((-2.0 * g + sq_col) + sq_row)
    negd = -dist
    feats = feats_ref[0]
    sumacc = jnp.zeros((_R, _NPROP), jnp.float32)
    maxacc = jnp.full((_R, _NPROP), _NEG, jnp.float32)
    taken = jnp.zeros((_R, 1), jnp.float32)
    for k in range(_K):
        m = jnp.max(negd, axis=1, keepdims=True)
        eq = negd == m
        onehot = jnp.where(eq, 1.0, 0.0)
        negd = jnp.where(eq, _NEG, negd)
        gke = jnp.dot(onehot, feats,
                      preferred_element_type=jnp.float32)
        if k == 0:
            taken = gke[:, _NPROP:_NPROP + 1] - 1.0
        else:
            gate = taken < float(_K - 1)
            taken = taken + gke[:, _NPROP:_NPROP + 1]
            w = jnp.exp(_EXPF * m)
            wk = w * gke[:, 0:_NPROP]
            sumacc = sumacc + jnp.where(gate, wk, 0.0)
            maxacc = jnp.maximum(maxacc, jnp.where(gate, wk, _NEG))
    mean = sumacc * (1.0 / (_K - 1))
    Wo = Wo_ref[...]
    out = (jnp.dot(x_rows.astype(jnp.bfloat16),
                   Wo[0:_F].astype(jnp.bfloat16),
                   preferred_element_type=jnp.float32)
           + jnp.dot(mean.astype(jnp.bfloat16),
                     Wo[_F:_F + _NPROP].astype(jnp.bfloat16),
                     preferred_element_type=jnp.float32)
           + jnp.dot(maxacc.astype(jnp.bfloat16),
                     Wo[_F + _NPROP:].astype(jnp.bfloat16),
                     preferred_element_type=jnp.float32)
           + bo_ref[...])
    out_ref[0] = out


def kernel(x, W_flr, b_flr, W_s, b_s, W_out, b_out):
    Ws_pad = jnp.zeros((_F, 128), jnp.float32).at[:, :_NDIM].set(W_s)
    bs_pad = jnp.zeros((1, 128), jnp.float32).at[:, :_NDIM].set(b_s)
    Wf_ext = jnp.zeros((_F, 128), jnp.float32).at[:, :_NPROP].set(W_flr)
    bf_ext = (jnp.zeros((1, 128), jnp.float32)
              .at[:, :_NPROP].set(b_flr.reshape(1, _NPROP))
              .at[:, _NPROP].set(1.0))
    bo = b_out.reshape(1, _NFILT)

    cpad, feats = pl.pallas_call(
        _xform_body,
        grid=(_B,),
        in_specs=[
            pl.BlockSpec((1, _V, _F), lambda b: (b, 0, 0)),
            pl.BlockSpec((_F, 128), lambda b: (0, 0)),
            pl.BlockSpec((1, 128), lambda b: (0, 0)),
            pl.BlockSpec((_F, 128), lambda b: (0, 0)),
            pl.BlockSpec((1, 128), lambda b: (0, 0)),
        ],
        out_specs=[
            pl.BlockSpec((1, _V, 128), lambda b: (b, 0, 0)),
            pl.BlockSpec((1, _V, 128), lambda b: (b, 0, 0)),
        ],
        out_shape=[
            jax.ShapeDtypeStruct((_B, _V, 128), jnp.float32),
            jax.ShapeDtypeStruct((_B, _V, 128), jnp.float32),
        ],
        compiler_params=pltpu.CompilerParams(
            dimension_semantics=("parallel",)),
    )(x, Ws_pad, bs_pad, Wf_ext, bf_ext)

    csl = cpad[:, :, :_NDIM]
    sq = jnp.sum(csl * csl, axis=2)
    sq_col = jnp.broadcast_to(sq[:, :, None], (_B, _V, 8))
    sq_row = jnp.broadcast_to(sq[:, None, :], (_B, 8, _V))

    return pl.pallas_call(
        _main_body,
        grid=(_B, _V // _R),
        in_specs=[
            pl.BlockSpec((1, _R, _F), lambda b, r: (b, r, 0)),
            pl.BlockSpec((1, _R, 128), lambda b, r: (b, r, 0)),
            pl.BlockSpec((1, _V, 128), lambda b, r: (b, 0, 0)),
            pl.BlockSpec((1, _R, 8), lambda b, r: (b, r, 0)),
            pl.BlockSpec((1, 8, _V), lambda b, r: (b, 0, 0)),
            pl.BlockSpec((1, _V, 128), lambda b, r: (b, 0, 0)),
            pl.BlockSpec((_F + 2 * _NPROP, _NFILT), lambda b, r: (0, 0)),
            pl.BlockSpec((1, _NFILT), lambda b, r: (0, 0)),
        ],
        out_specs=pl.BlockSpec((1, _R, _NFILT), lambda b, r: (b, r, 0)),
        out_shape=jax.ShapeDtypeStruct((_B, _V, _NFILT), jnp.float32),
        compiler_params=pltpu.CompilerParams(
            dimension_semantics=("parallel", "arbitrary")),
    )(x, cpad, cpad, sq_col, sq_row, feats, W_out, bo)


# speedup vs baseline: 1.4864x; 1.0738x over previous
import jax
import jax.numpy as jnp
from jax.experimental import pallas as pl
from jax.experimental.pallas import tpu as pltpu

_B, _V, _F = 8, 2048, 64
_K = 16
_NDIM = 4
_NPROP = 64
_NFILT = 128
_EXPF = 10.0
_R = 256
_NEG = float("-inf")


def _xform_body(x_ref, Ws_ref, bs_ref, Wf_ref, bf_ref, c_ref, f_ref):
    xb = x_ref[0].astype(jnp.bfloat16)
    c_ref[0] = (jnp.dot(xb, Ws_ref[...].astype(jnp.bfloat16),
                        preferred_element_type=jnp.float32) + bs_ref[...])
    f_ref[0] = (jnp.dot(xb, Wf_ref[...].astype(jnp.bfloat16),
                        preferred_element_type=jnp.float32) + bf_ref[...])


def _main_body(x_rows_ref, c_rows_ref, c_all_ref, sqc_ref, sqr_ref,
               feats_ref, Wo_ref, bo_ref, out_ref):
    x_rows = x_rows_ref[0]
    c_rows = c_rows_ref[0]
    c_all = c_all_ref[0]
    sq_col = sqc_ref[0][:, 0:1]
    sq_row = sqr_ref[0][0:1, :]
    g = jax.lax.dot_general(c_rows.astype(jnp.bfloat16),
                            c_all.astype(jnp.bfloat16),
                            (((1,), (1,)), ((), ())),
                            preferred_element_type=jnp.float32)
    dist = jnp.abs((-2.0 * g + sq_col) + sq_row)
    negd = -dist
    feats = feats_ref[0]
    sumacc = jnp.zeros((_R, _NPROP), jnp.float32)
    maxacc = jnp.full((_R, _NPROP), _NEG, jnp.float32)
    taken = jnp.zeros((_R, 1), jnp.float32)
    for k in range(_K):
        m = jnp.max(negd, axis=1, keepdims=True)
        eq = negd == m
        onehot = jnp.where(eq, 1.0, 0.0)
        negd = jnp.where(eq, _NEG, negd)
        gke = jnp.dot(onehot, feats,
                      preferred_element_type=jnp.float32)
        if k == 0:
            taken = gke[:, _NPROP:_NPROP + 1] - 1.0
        else:
            gate = taken < float(_K - 1)
            taken = taken + gke[:, _NPROP:_NPROP + 1]
            w = jnp.exp(_EXPF * m)
            wk = w * gke[:, 0:_NPROP]
            sumacc = sumacc + jnp.where(gate, wk, 0.0)
            maxacc = jnp.maximum(maxacc, jnp.where(gate, wk, _NEG))
    mean = sumacc * (1.0 / (_K - 1))
    Wo = Wo_ref[...]
    out = (jnp.dot(x_rows.astype(jnp.bfloat16),
                   Wo[0:_F].astype(jnp.bfloat16),
                   preferred_element_type=jnp.float32)
           + jnp.dot(mean.astype(jnp.bfloat16),
                     Wo[_F:_F + _NPROP].astype(jnp.bfloat16),
                     preferred_element_type=jnp.float32)
           + jnp.dot(maxacc.astype(jnp.bfloat16),
                     Wo[_F + _NPROP:].astype(jnp.bfloat16),
                     preferred_element_type=jnp.float32)
           + bo_ref[...])
    out_ref[0] = out


def kernel(x, W_flr, b_flr, W_s, b_s, W_out, b_out):
    Ws_pad = jnp.zeros((_F, 128), jnp.float32).at[:, :_NDIM].set(W_s)
    bs_pad = jnp.zeros((1, 128), jnp.float32).at[:, :_NDIM].set(b_s)
    Wf_ext = jnp.zeros((_F, 128), jnp.float32).at[:, :_NPROP].set(W_flr)
    bf_ext = (jnp.zeros((1, 128), jnp.float32)
              .at[:, :_NPROP].set(b_flr.reshape(1, _NPROP))
              .at[:, _NPROP].set(1.0))
    bo = b_out.reshape(1, _NFILT)

    cpad, feats = pl.pallas_call(
        _xform_body,
        grid=(_B,),
        in_specs=[
            pl.BlockSpec((1, _V, _F), lambda b: (b, 0, 0)),
            pl.BlockSpec((_F, 128), lambda b: (0, 0)),
            pl.BlockSpec((1, 128), lambda b: (0, 0)),
            pl.BlockSpec((_F, 128), lambda b: (0, 0)),
            pl.BlockSpec((1, 128), lambda b: (0, 0)),
        ],
        out_specs=[
            pl.BlockSpec((1, _V, 128), lambda b: (b, 0, 0)),
            pl.BlockSpec((1, _V, 128), lambda b: (b, 0, 0)),
        ],
        out_shape=[
            jax.ShapeDtypeStruct((_B, _V, 128), jnp.float32),
            jax.ShapeDtypeStruct((_B, _V, 128), jnp.float32),
        ],
        compiler_params=pltpu.CompilerParams(
            dimension_semantics=("parallel",)),
    )(x, Ws_pad, bs_pad, Wf_ext, bf_ext)

    csl = cpad[:, :, :_NDIM]
    sq = jnp.sum(csl * csl, axis=2)
    sq_col = jnp.broadcast_to(sq[:, :, None], (_B, _V, 8))
    sq_row = jnp.broadcast_to(sq[:, None, :], (_B, 8, _V))

    return pl.pallas_call(
        _main_body,
        grid=(_B, _V // _R),
        in_specs=[
            pl.BlockSpec((1, _R, _F), lambda b, r: (b, r, 0)),
            pl.BlockSpec((1, _R, 128), lambda b, r: (b, r, 0)),
            pl.BlockSpec((1, _V, 128), lambda b, r: (b, 0, 0)),
            pl.BlockSpec((1, _R, 8), lambda b, r: (b, r, 0)),
            pl.BlockSpec((1, 8, _V), lambda b, r: (b, 0, 0)),
            pl.BlockSpec((1, _V, 128), lambda b, r: (b, 0, 0)),
            pl.BlockSpec((_F + 2 * _NPROP, _NFILT), lambda b, r: (0, 0)),
            pl.BlockSpec((1, _NFILT), lambda b, r: (0, 0)),
        ],
        out_specs=pl.BlockSpec((1, _R, _NFILT), lambda b, r: (b, r, 0)),
        out_shape=jax.ShapeDtypeStruct((_B, _V, _NFILT), jnp.float32),
        compiler_params=pltpu.CompilerParams(
            dimension_semantics=("parallel", "arbitrary")),
    )(x, cpad, cpad, sq_col, sq_row, feats, W_out, bo)
